# baseline (speedup 1.0000x reference)
import functools

import jax
import jax.numpy as jnp
from jax import lax
from jax.experimental import pallas as pl
from jax.experimental.pallas import tpu as pltpu
from jax.experimental.pallas import tpu_sc as plsc

BATCH = 16384
HIST = 50
EMB = 32
ROW = 128
OW = 64


def _tca_tab(table_t):
    V = table_t.shape[1]
    TBLK = 8192
    grid = (pl.cdiv(V, TBLK),)

    def body(tin, tout):
        eye = (lax.broadcasted_iota(jnp.int32, (EMB, EMB), 0)
               == lax.broadcasted_iota(jnp.int32, (EMB, EMB), 1)
               ).astype(jnp.float32)
        t = lax.dot_general(tin[...], eye, (((0,), (0,)), ((), ())),
                            preferred_element_type=jnp.float32)
        t = t.reshape(TBLK // 4, 4, EMB)
        for j in range(4):
            tout[:, EMB * j:EMB * (j + 1)] = t[:, j, :]

    return pl.pallas_call(
        body,
        grid=grid,
        in_specs=[pl.BlockSpec((EMB, TBLK), lambda i: (0, i))],
        out_specs=pl.BlockSpec((TBLK // 4, ROW), lambda i: (i, 0)),
        out_shape=jax.ShapeDtypeStruct((V // 4, ROW), jnp.float32),
    )(table_t)


def _tca_idx(xt):
    XBLK = 4096
    grid = (BATCH // XBLK,)

    def body(tin, tout):
        tout[:, :HIST] = tin[...].T

    return pl.pallas_call(
        body,
        grid=grid,
        in_specs=[pl.BlockSpec((HIST, XBLK), lambda i: (0, i))],
        out_specs=pl.BlockSpec((XBLK, ROW), lambda i: (i, 0)),
        out_shape=jax.ShapeDtypeStruct((BATCH, ROW), jnp.int32),
    )(xt)


def _tcc(outp):
    BB = 512
    RPB = HIST * OW // ROW
    grid = (BATCH // BB,)

    def body(tin, tout):
        t = tin[...].reshape(BB, HIST * OW).T
        tout[...] = t.reshape(HIST, OW, BB)[:, :EMB, :]

    return pl.pallas_call(
        body,
        grid=grid,
        in_specs=[pl.BlockSpec((BB * RPB, ROW), lambda i: (i, 0))],
        out_specs=pl.BlockSpec((HIST, EMB, BB), lambda i: (0, 0, i)),
        out_shape=jax.ShapeDtypeStruct((HIST, EMB, BATCH), jnp.float32),
    )(outp)


def _make_scb(V):
    info = plsc.get_sparse_core_info()
    NC, NS = info.num_cores, info.num_subcores
    NW = NC * NS
    b_per_w = BATCH // NW
    XST = 2
    xb_st = b_per_w // XST
    n_st = xb_st * HIST
    CH = 128
    GC = 4
    GROUP = CH * GC
    NBUF = 2
    n_groups = n_st // GROUP

    mesh = plsc.VectorSubcoreMesh(core_axis_name="c", subcore_axis_name="s")

    @functools.partial(
        pl.kernel,
        mesh=mesh,
        compiler_params=pltpu.CompilerParams(
            use_tc_tiling_on_sc=False, needs_layout_passes=False),
        out_type=jax.ShapeDtypeStruct((BATCH * HIST, OW), jnp.float32),
        scratch_types=[
            pltpu.VMEM((xb_st, ROW), jnp.int32),
            pltpu.VMEM((n_st,), jnp.int32),
            pltpu.VMEM((NBUF, GROUP, EMB), jnp.float32),
            pltpu.SemaphoreType.DMA((NBUF,)),
            pltpu.SemaphoreType.DMA((NBUF,)),
        ],
    )
    def scb(xb_hbm, tab_hbm, out_hbm, xv, xf, rows_v, gsem, ssem):
        cid = lax.axis_index("c")
        sid = lax.axis_index("s")
        wid = cid * NS + sid
        base = wid * b_per_w
        lanes = lax.iota(jnp.int32, 16)

        def x_stage(st, carry):
            pltpu.sync_copy(xb_hbm.at[pl.ds(base + st * xb_st, xb_st)], xv)

            def tr_body(k, carry2):
                s = lanes + k * 16
                vals = plsc.load_gather(xv, [s // HIST, s % HIST])
                xf[pl.ds(k * 16, 16)] = vals
                return carry2

            lax.fori_loop(0, n_st // 16, tr_body, 0)

            row0 = (base + st * xb_st) * HIST

            def issue_gathers(g, b):
                for j in range(GC):
                    pltpu.async_copy(
                        tab_hbm.at[xf.at[pl.ds(g * GROUP + j * CH, CH)]],
                        rows_v.at[b, pl.ds(j * CH, CH)],
                        gsem.at[b],
                    )

            def wait_gathers(g, b):
                for j in range(GC):
                    pltpu.make_async_copy(
                        tab_hbm.at[xf.at[pl.ds(g * GROUP + j * CH, CH)]],
                        rows_v.at[b, pl.ds(j * CH, CH)],
                        gsem.at[b],
                    ).wait()

            def wait_store(b):
                pltpu.make_async_copy(
                    rows_v.at[b],
                    out_hbm.at[pl.ds(0, GROUP), pl.ds(0, EMB)],
                    ssem.at[b],
                ).wait()

            issue_gathers(0, 0)

            def group_body(g, carry2):
                b = g % NBUF
                wait_gathers(g, b)
                pltpu.async_copy(
                    rows_v.at[b],
                    out_hbm.at[pl.ds(row0 + g * GROUP, GROUP),
                               pl.ds(0, EMB)],
                    ssem.at[b],
                )
                gn = g + 1
                bn = gn % NBUF

                @pl.when(jnp.logical_and(gn < n_groups, g >= 1))
                def _():
                    wait_store(bn)

                @pl.when(gn < n_groups)
                def _():
                    issue_gathers(gn, bn)

                return carry2

            lax.fori_loop(0, n_groups, group_body, 0)

            for t in range(n_groups - NBUF, n_groups):
                wait_store(t % NBUF)
            return carry

        lax.fori_loop(0, XST, x_stage, 0)

    return scb


def kernel(x, table):
    V, D = table.shape
    xt = x.astype(jnp.int32).T
    tab2d = _tca_tab(table.T)
    xb = _tca_idx(xt)
    tab_lin = tab2d.reshape(V, EMB)
    out64 = _make_scb(V)(xb, tab_lin)
    out_t = _tcc(out64.reshape(BATCH * HIST * OW // ROW, ROW))
    return out_t.transpose(2, 0, 1)


# speedup vs baseline: 1.0565x; 1.0565x over previous
import functools

import jax
import jax.numpy as jnp
from jax import lax
from jax.experimental import pallas as pl
from jax.experimental.pallas import tpu as pltpu
from jax.experimental.pallas import tpu_sc as plsc

BATCH = 16384
HIST = 50
EMB = 32
ROW = 128
OW = 64


def _tca_tab(table_t):
    V = table_t.shape[1]
    TBLK = 16384
    grid = (pl.cdiv(V, TBLK),)

    def body(tin, tout):
        t = tin[...].T.reshape(TBLK // 4, 4, EMB)
        for j in range(4):
            tout[:, EMB * j:EMB * (j + 1)] = t[:, j, :]

    return pl.pallas_call(
        body,
        grid=grid,
        in_specs=[pl.BlockSpec((EMB, TBLK), lambda i: (0, i))],
        out_specs=pl.BlockSpec((TBLK // 4, ROW), lambda i: (i, 0)),
        out_shape=jax.ShapeDtypeStruct((V // 4, ROW), jnp.float32),
    )(table_t)


def _tca_idx(xt):
    XBLK = 4096
    grid = (BATCH // XBLK,)

    def body(tin, tout):
        tout[:, :HIST] = tin[...].T

    return pl.pallas_call(
        body,
        grid=grid,
        in_specs=[pl.BlockSpec((HIST, XBLK), lambda i: (0, i))],
        out_specs=pl.BlockSpec((XBLK, ROW), lambda i: (i, 0)),
        out_shape=jax.ShapeDtypeStruct((BATCH, ROW), jnp.int32),
    )(xt)


def _tcc(outp):
    BB = 512
    RPB = HIST * OW // ROW
    grid = (BATCH // BB,)

    def body(tin, tout):
        t = tin[...].reshape(BB, HIST * OW).T
        tout[...] = t.reshape(HIST, OW, BB)[:, :EMB, :]

    return pl.pallas_call(
        body,
        grid=grid,
        in_specs=[pl.BlockSpec((BB * RPB, ROW), lambda i: (i, 0))],
        out_specs=pl.BlockSpec((HIST, EMB, BB), lambda i: (0, 0, i)),
        out_shape=jax.ShapeDtypeStruct((HIST, EMB, BATCH), jnp.float32),
    )(outp)


def _make_scb(V):
    info = plsc.get_sparse_core_info()
    NC, NS = info.num_cores, info.num_subcores
    NW = NC * NS
    b_per_w = BATCH // NW
    XST = 2
    xb_st = b_per_w // XST
    n_st = xb_st * HIST
    CH = 128
    GC = 4
    GROUP = CH * GC
    NBUF = 2
    n_groups = n_st // GROUP

    mesh = plsc.VectorSubcoreMesh(core_axis_name="c", subcore_axis_name="s")

    @functools.partial(
        pl.kernel,
        mesh=mesh,
        compiler_params=pltpu.CompilerParams(
            use_tc_tiling_on_sc=False, needs_layout_passes=False),
        out_type=jax.ShapeDtypeStruct((BATCH * HIST, OW), jnp.float32),
        scratch_types=[
            pltpu.VMEM((xb_st, ROW), jnp.int32),
            pltpu.VMEM((n_st,), jnp.int32),
            pltpu.VMEM((NBUF, GROUP, EMB), jnp.float32),
            pltpu.SemaphoreType.DMA((NBUF,)),
            pltpu.SemaphoreType.DMA((NBUF,)),
        ],
    )
    def scb(xb_hbm, tab_hbm, out_hbm, xv, xf, rows_v, gsem, ssem):
        cid = lax.axis_index("c")
        sid = lax.axis_index("s")
        wid = cid * NS + sid
        base = wid * b_per_w
        lanes = lax.iota(jnp.int32, 16)

        def x_stage(st, carry):
            pltpu.sync_copy(xb_hbm.at[pl.ds(base + st * xb_st, xb_st)], xv)

            def tr_body(k, carry2):
                s = lanes + k * 16
                vals = plsc.load_gather(xv, [s // HIST, s % HIST])
                xf[pl.ds(k * 16, 16)] = vals
                return carry2

            lax.fori_loop(0, n_st // 16, tr_body, 0)

            row0 = (base + st * xb_st) * HIST

            def issue_gathers(g, b):
                for j in range(GC):
                    pltpu.async_copy(
                        tab_hbm.at[xf.at[pl.ds(g * GROUP + j * CH, CH)]],
                        rows_v.at[b, pl.ds(j * CH, CH)],
                        gsem.at[b],
                    )

            def wait_gathers(g, b):
                for j in range(GC):
                    pltpu.make_async_copy(
                        tab_hbm.at[xf.at[pl.ds(g * GROUP + j * CH, CH)]],
                        rows_v.at[b, pl.ds(j * CH, CH)],
                        gsem.at[b],
                    ).wait()

            def wait_store(b):
                pltpu.make_async_copy(
                    rows_v.at[b],
                    out_hbm.at[pl.ds(0, GROUP), pl.ds(0, EMB)],
                    ssem.at[b],
                ).wait()

            issue_gathers(0, 0)

            def group_body(g, carry2):
                b = g % NBUF
                wait_gathers(g, b)
                pltpu.async_copy(
                    rows_v.at[b],
                    out_hbm.at[pl.ds(row0 + g * GROUP, GROUP),
                               pl.ds(0, EMB)],
                    ssem.at[b],
                )
                gn = g + 1
                bn = gn % NBUF

                @pl.when(jnp.logical_and(gn < n_groups, g >= 1))
                def _():
                    wait_store(bn)

                @pl.when(gn < n_groups)
                def _():
                    issue_gathers(gn, bn)

                return carry2

            lax.fori_loop(0, n_groups, group_body, 0)

            for t in range(n_groups - NBUF, n_groups):
                wait_store(t % NBUF)
            return carry

        lax.fori_loop(0, XST, x_stage, 0)

    return scb


def kernel(x, table):
    V, D = table.shape
    xt = x.astype(jnp.int32).T
    tab2d = _tca_tab(table.T)
    xb = _tca_idx(xt)
    tab_lin = tab2d.reshape(V, EMB)
    out64 = _make_scb(V)(xb, tab_lin)
    out_t = _tcc(out64.reshape(BATCH * HIST * OW // ROW, ROW))
    return out_t.transpose(2, 0, 1)
